# baseline (speedup 1.0000x reference)
import functools

import jax
import jax.numpy as jnp
from jax import lax
from jax.experimental import pallas as pl
from jax.experimental.pallas import tpu as pltpu
from jax.experimental.pallas import tpu_sc as plsc

D = 64
LANE = 128
K = 4
CHUNK = K * LANE
VB = 4096
NPAIR = 123 * VB
HALF1_BASE = 122 * VB


def _conv1_body(x1_ref, x2_ref, o_ref):
    o_ref[:, 0:D] = x1_ref[...].T
    o_ref[:, D : 2 * D] = x2_ref[...].T


def _repack_table(tT):
    return pl.pallas_call(
        _conv1_body,
        grid=(NPAIR // VB,),
        in_specs=[
            pl.BlockSpec((D, VB), lambda i: (0, i)),
            pl.BlockSpec((D, VB), lambda i: (0, HALF1_BASE // VB + i)),
        ],
        out_specs=pl.BlockSpec((VB, 128), lambda i: (i, 0)),
        out_shape=jax.ShapeDtypeStruct((NPAIR, 128), jnp.float32),
    )(tT, tT)


@functools.partial(jax.jit, static_argnums=(0, 1))
def _gather_call(BT, S, idx2d, table128):
    B = BT * S
    info = plsc.get_sparse_core_info()
    nw = info.num_cores * info.num_subcores
    assert B % (nw * CHUNK) == 0 and BT % CHUNK == 0
    n_chunks = B // (nw * CHUNK)
    mesh = plsc.VectorSubcoreMesh(core_axis_name="c", subcore_axis_name="s")

    @functools.partial(
        pl.kernel,
        mesh=mesh,
        out_type=jax.ShapeDtypeStruct((S, D, BT), jnp.float32),
        scratch_types=[
            pltpu.VMEM((K, LANE), jnp.int32),
            pltpu.VMEM((CHUNK,), jnp.int32),
            pltpu.VMEM((CHUNK, 128), jnp.float32),
            pltpu.VMEM((D, CHUNK), jnp.float32),
            pltpu.SemaphoreType.DMA,
        ],
        compiler_params=pltpu.CompilerParams(
            use_tc_tiling_on_sc=False, needs_layout_passes=False
        ),
    )
    def gather_kernel(
        idx_hbm, table_hbm, out_hbm, idx_v, hcol_v, rows_v, t_v, sem
    ):
        wid = lax.axis_index("s") * info.num_cores + lax.axis_index("c")
        r_base = wid * (n_chunks * CHUNK)
        lanes = lax.iota(jnp.int32, 16)

        def chunk_body(i, carry):
            r0 = r_base + i * CHUNK
            pltpu.sync_copy(idx_hbm.at[pl.ds(r0 // LANE, K)], idx_v)
            for j in range(K):
                for k in range(LANE // 16):
                    v = idx_v[j, pl.ds(k * 16, 16)]
                    in0 = v < NPAIR
                    idx_v[j, pl.ds(k * 16, 16)] = jnp.where(
                        in0, v, v - HALF1_BASE
                    )
                    hcol_v[pl.ds(j * LANE + k * 16, 16)] = jnp.where(
                        in0, 0, D
                    )
            cps = [
                pltpu.async_copy(
                    table_hbm.at[idx_v.at[j]],
                    rows_v.at[pl.ds(j * LANE, LANE)],
                    sem,
                )
                for j in range(K)
            ]
            for cp in cps:
                cp.wait()

            def tbody(g4, carry2):
                for gg in range(4):
                    g = g4 * 4 + gg
                    rowv = g * 16 + lanes
                    colb = hcol_v[pl.ds(g * 16, 16)]
                    for d in range(D):
                        vals = plsc.load_gather(rows_v, [rowv, colb + d])
                        t_v[d, pl.ds(g * 16, 16)] = vals
                return carry2

            lax.fori_loop(0, CHUNK // 64, tbody, 0)

            s_slot = r0 // BT
            b0 = r0 % BT
            pltpu.sync_copy(t_v, out_hbm.at[s_slot, :, pl.ds(b0, CHUNK)])
            return carry

        lax.fori_loop(0, n_chunks, chunk_body, 0)

    return gather_kernel(idx2d, table128)


def kernel(tokens, tok_embeddings):
    bt, s = tokens.shape
    B = bt * s
    table128 = _repack_table(tok_embeddings.T.astype(jnp.float32))
    idx2d = tokens.T.astype(jnp.int32).reshape(B // LANE, LANE)
    res = _gather_call(bt, s, idx2d, table128)
    return res.transpose(2, 0, 1)


# speedup vs baseline: 2.1180x; 2.1180x over previous
import functools

import jax
import jax.numpy as jnp
from jax import lax
from jax.experimental import pallas as pl
from jax.experimental.pallas import tpu as pltpu
from jax.experimental.pallas import tpu_sc as plsc

D = 64
LANE = 128
K = 8
CHUNK = K * LANE
VB = 4096
NPAIR = 123 * VB
HALF1_BASE = 122 * VB


def _conv1_body(x1_ref, x2_ref, o_ref):
    o_ref[:, 0:D] = x1_ref[...].T
    o_ref[:, D : 2 * D] = x2_ref[...].T


def _repack_table(tT):
    return pl.pallas_call(
        _conv1_body,
        grid=(NPAIR // VB,),
        in_specs=[
            pl.BlockSpec((D, VB), lambda i: (0, i)),
            pl.BlockSpec((D, VB), lambda i: (0, HALF1_BASE // VB + i)),
        ],
        out_specs=pl.BlockSpec((VB, 128), lambda i: (i, 0)),
        out_shape=jax.ShapeDtypeStruct((NPAIR, 128), jnp.float32),
    )(tT, tT)


def _conv2_body(x_ref, o_ref):
    bt = o_ref.shape[2]
    o_ref[0, :, 0 : bt // 2] = x_ref[:, 0:D].T
    o_ref[0, :, bt // 2 : bt] = x_ref[:, D : 2 * D].T


def _transpose_out(view, S, BT):
    return pl.pallas_call(
        _conv2_body,
        grid=(S,),
        in_specs=[pl.BlockSpec((BT // 2, 128), lambda s: (s, 0))],
        out_specs=pl.BlockSpec((1, D, BT), lambda s: (s, 0, 0)),
        out_shape=jax.ShapeDtypeStruct((S, D, BT), jnp.float32),
    )(view)


@functools.partial(jax.jit, static_argnums=(0,))
def _gather_call(B, idx2d, table):
    info = plsc.get_sparse_core_info()
    nw = info.num_cores * info.num_subcores
    assert B % (nw * CHUNK) == 0
    n_chunks = B // (nw * CHUNK)
    mesh = plsc.VectorSubcoreMesh(core_axis_name="c", subcore_axis_name="s")

    @functools.partial(
        pl.kernel,
        mesh=mesh,
        out_type=jax.ShapeDtypeStruct((B, D), jnp.float32),
        scratch_types=[
            pltpu.VMEM((K, LANE), jnp.int32),
            pltpu.VMEM((CHUNK, D), jnp.float32),
            pltpu.SemaphoreType.DMA,
        ],
        compiler_params=pltpu.CompilerParams(use_tc_tiling_on_sc=False),
    )
    def gather_kernel(idx_hbm, table_hbm, out_hbm, idx_v, rows_v, sem):
        wid = lax.axis_index("s") * info.num_cores + lax.axis_index("c")
        idx_row0 = wid * (n_chunks * K)
        out_row0 = wid * (n_chunks * CHUNK)

        def chunk_body(i, carry):
            pltpu.sync_copy(idx_hbm.at[pl.ds(idx_row0 + i * K, K)], idx_v)
            for j in range(K):
                for k in range(LANE // 16):
                    v = idx_v[j, pl.ds(k * 16, 16)]
                    v2 = v + v
                    idx_v[j, pl.ds(k * 16, 16)] = jnp.where(
                        v < NPAIR, v2, v2 - (2 * HALF1_BASE - 1)
                    )
            cps = [
                pltpu.async_copy(
                    table_hbm.at[idx_v.at[j]],
                    rows_v.at[pl.ds(j * LANE, LANE)],
                    sem,
                )
                for j in range(K)
            ]
            for cp in cps:
                cp.wait()
            pltpu.sync_copy(
                rows_v, out_hbm.at[pl.ds(out_row0 + i * CHUNK, CHUNK)]
            )
            return carry

        lax.fori_loop(0, n_chunks, chunk_body, 0)

    return gather_kernel(idx2d, table)


def kernel(tokens, tok_embeddings):
    bt, s = tokens.shape
    B = bt * s
    table_rm = _repack_table(tok_embeddings.T.astype(jnp.float32))
    idx2d = (
        tokens.T.astype(jnp.int32)
        .reshape(s, 2, bt // 2)
        .swapaxes(1, 2)
        .reshape(B // LANE, LANE)
    )
    out_rm = _gather_call(B, idx2d, table_rm.reshape(2 * NPAIR, D))
    res = _transpose_out(out_rm.reshape(B // 2, 128), s, bt)
    return res.transpose(2, 0, 1)


# speedup vs baseline: 2.1826x; 1.0305x over previous
import functools

import jax
import jax.numpy as jnp
from jax import lax
from jax.experimental import pallas as pl
from jax.experimental.pallas import tpu as pltpu
from jax.experimental.pallas import tpu_sc as plsc

D = 64
LANE = 128
K = 8
CHUNK = K * LANE
VB = 8192
NPAIR = 62 * VB
HALF1_BASE = 61 * VB


def _conv1_body(x1_ref, x2_ref, o_ref):
    o_ref[:, 0:D] = x1_ref[...].T
    o_ref[:, D : 2 * D] = x2_ref[...].T


def _repack_table(tT):
    return pl.pallas_call(
        _conv1_body,
        grid=(NPAIR // VB,),
        in_specs=[
            pl.BlockSpec((D, VB), lambda i: (0, i)),
            pl.BlockSpec((D, VB), lambda i: (0, HALF1_BASE // VB + i)),
        ],
        out_specs=pl.BlockSpec((VB, 128), lambda i: (i, 0)),
        out_shape=jax.ShapeDtypeStruct((NPAIR, 128), jnp.float32),
    )(tT, tT)


def _conv2_body(x_ref, o_ref):
    bt = o_ref.shape[2]
    o_ref[0, :, 0 : bt // 2] = x_ref[:, 0:D].T
    o_ref[0, :, bt // 2 : bt] = x_ref[:, D : 2 * D].T


def _transpose_out(view, S, BT):
    return pl.pallas_call(
        _conv2_body,
        grid=(S,),
        in_specs=[pl.BlockSpec((BT // 2, 128), lambda s: (s, 0))],
        out_specs=pl.BlockSpec((1, D, BT), lambda s: (s, 0, 0)),
        out_shape=jax.ShapeDtypeStruct((S, D, BT), jnp.float32),
    )(view)


@functools.partial(jax.jit, static_argnums=(0,))
def _gather_call(B, idx2d, table):
    info = plsc.get_sparse_core_info()
    nw = info.num_cores * info.num_subcores
    assert B % (nw * CHUNK) == 0
    n_chunks = B // (nw * CHUNK)
    mesh = plsc.VectorSubcoreMesh(core_axis_name="c", subcore_axis_name="s")

    @functools.partial(
        pl.kernel,
        mesh=mesh,
        out_type=jax.ShapeDtypeStruct((B, D), jnp.float32),
        scratch_types=[
            pltpu.VMEM((K, LANE), jnp.int32),
            pltpu.VMEM((CHUNK, D), jnp.float32),
            pltpu.SemaphoreType.DMA,
        ],
        compiler_params=pltpu.CompilerParams(use_tc_tiling_on_sc=False),
    )
    def gather_kernel(idx_hbm, table_hbm, out_hbm, idx_v, rows_v, sem):
        wid = lax.axis_index("s") * info.num_cores + lax.axis_index("c")
        idx_row0 = wid * (n_chunks * K)
        out_row0 = wid * (n_chunks * CHUNK)

        def chunk_body(i, carry):
            pltpu.sync_copy(idx_hbm.at[pl.ds(idx_row0 + i * K, K)], idx_v)
            for j in range(K):
                for k in range(LANE // 16):
                    v = idx_v[j, pl.ds(k * 16, 16)]
                    v2 = v + v
                    idx_v[j, pl.ds(k * 16, 16)] = jnp.where(
                        v < NPAIR, v2, v2 - (2 * HALF1_BASE - 1)
                    )
            cps = [
                pltpu.async_copy(
                    table_hbm.at[idx_v.at[j]],
                    rows_v.at[pl.ds(j * LANE, LANE)],
                    sem,
                )
                for j in range(K)
            ]
            for cp in cps:
                cp.wait()
            pltpu.sync_copy(
                rows_v, out_hbm.at[pl.ds(out_row0 + i * CHUNK, CHUNK)]
            )
            return carry

        lax.fori_loop(0, n_chunks, chunk_body, 0)

    return gather_kernel(idx2d, table)


def kernel(tokens, tok_embeddings):
    bt, s = tokens.shape
    B = bt * s
    table_rm = _repack_table(tok_embeddings.T.astype(jnp.float32))
    idx2d = (
        tokens.T.astype(jnp.int32)
        .reshape(s, 2, bt // 2)
        .swapaxes(1, 2)
        .reshape(B // LANE, LANE)
    )
    out_rm = _gather_call(B, idx2d, table_rm.reshape(2 * NPAIR, D))
    res = _transpose_out(out_rm.reshape(B // 2, 128), s, bt)
    return res.transpose(2, 0, 1)


# speedup vs baseline: 2.4074x; 1.1030x over previous
import functools

import jax
import jax.numpy as jnp
from jax import lax
from jax.experimental import pallas as pl
from jax.experimental.pallas import tpu as pltpu
from jax.experimental.pallas import tpu_sc as plsc

D = 64
LANE = 128
K = 8
CHUNK = K * LANE
VB = 8192
NPAIR = 62 * VB
HALF1_BASE = 61 * VB


def _conv1_body(x1_ref, x2_ref, o_ref):
    o_ref[:, 0:D] = x1_ref[...].T
    o_ref[:, D : 2 * D] = x2_ref[...].T


def _repack_table(tT):
    return pl.pallas_call(
        _conv1_body,
        grid=(NPAIR // VB,),
        in_specs=[
            pl.BlockSpec((D, VB), lambda i: (0, i)),
            pl.BlockSpec((D, VB), lambda i: (0, HALF1_BASE // VB + i)),
        ],
        out_specs=pl.BlockSpec((VB, 128), lambda i: (i, 0)),
        out_shape=jax.ShapeDtypeStruct((NPAIR, 128), jnp.float32),
    )(tT, tT)


SLOTS = 4


def _conv2_body(x_ref, o_ref):
    bt = o_ref.shape[2]
    h = bt // 2
    for q in range(SLOTS):
        o_ref[q, :, 0:h] = x_ref[pl.ds(q * h, h), 0:D].T
        o_ref[q, :, h:bt] = x_ref[pl.ds(q * h, h), D : 2 * D].T


def _transpose_out(view, S, BT):
    return pl.pallas_call(
        _conv2_body,
        grid=(S // SLOTS,),
        in_specs=[
            pl.BlockSpec((SLOTS * BT // 2, 128), lambda s: (s, 0))
        ],
        out_specs=pl.BlockSpec((SLOTS, D, BT), lambda s: (s, 0, 0)),
        out_shape=jax.ShapeDtypeStruct((S, D, BT), jnp.float32),
    )(view)


@functools.partial(jax.jit, static_argnums=(0,))
def _gather_call(B, idx2d, table):
    info = plsc.get_sparse_core_info()
    nw = info.num_cores * info.num_subcores
    assert B % (nw * CHUNK) == 0
    n_chunks = B // (nw * CHUNK)
    mesh = plsc.VectorSubcoreMesh(core_axis_name="c", subcore_axis_name="s")

    @functools.partial(
        pl.kernel,
        mesh=mesh,
        out_type=jax.ShapeDtypeStruct((B, D), jnp.float32),
        scratch_types=[
            pltpu.VMEM((K, LANE), jnp.int32),
            pltpu.VMEM((CHUNK, D), jnp.float32),
            pltpu.SemaphoreType.DMA,
        ],
        compiler_params=pltpu.CompilerParams(use_tc_tiling_on_sc=False),
    )
    def gather_kernel(idx_hbm, table_hbm, out_hbm, idx_v, rows_v, sem):
        wid = lax.axis_index("s") * info.num_cores + lax.axis_index("c")
        idx_row0 = wid * (n_chunks * K)
        out_row0 = wid * (n_chunks * CHUNK)

        def chunk_body(i, carry):
            pltpu.sync_copy(idx_hbm.at[pl.ds(idx_row0 + i * K, K)], idx_v)
            for j in range(K):
                for k in range(LANE // 16):
                    v = idx_v[j, pl.ds(k * 16, 16)]
                    v2 = v + v
                    idx_v[j, pl.ds(k * 16, 16)] = jnp.where(
                        v < NPAIR, v2, v2 - (2 * HALF1_BASE - 1)
                    )
            cps = [
                pltpu.async_copy(
                    table_hbm.at[idx_v.at[j]],
                    rows_v.at[pl.ds(j * LANE, LANE)],
                    sem,
                )
                for j in range(K)
            ]
            for cp in cps:
                cp.wait()
            pltpu.sync_copy(
                rows_v, out_hbm.at[pl.ds(out_row0 + i * CHUNK, CHUNK)]
            )
            return carry

        lax.fori_loop(0, n_chunks, chunk_body, 0)

    return gather_kernel(idx2d, table)


def kernel(tokens, tok_embeddings):
    bt, s = tokens.shape
    B = bt * s
    table_rm = _repack_table(tok_embeddings.T.astype(jnp.float32))
    idx2d = (
        tokens.T.astype(jnp.int32)
        .reshape(s, 2, bt // 2)
        .swapaxes(1, 2)
        .reshape(B // LANE, LANE)
    )
    out_rm = _gather_call(B, idx2d, table_rm.reshape(2 * NPAIR, D))
    res = _transpose_out(out_rm.reshape(B // 2, 128), s, bt)
    return res.transpose(2, 0, 1)


# speedup vs baseline: 2.4439x; 1.0152x over previous
import functools

import jax
import jax.numpy as jnp
from jax import lax
from jax.experimental import pallas as pl
from jax.experimental.pallas import tpu as pltpu
from jax.experimental.pallas import tpu_sc as plsc

D = 64
LANE = 128
K = 8
CHUNK = K * LANE
VB = 8192
NPAIR = 62 * VB
HALF1_BASE = 61 * VB


def _conv1_body(x1_ref, x2_ref, o_ref):
    o_ref[:, 0:D] = x1_ref[...].T
    o_ref[:, D : 2 * D] = x2_ref[...].T


def _repack_table(tT):
    return pl.pallas_call(
        _conv1_body,
        grid=(NPAIR // VB,),
        in_specs=[
            pl.BlockSpec((D, VB), lambda i: (0, i)),
            pl.BlockSpec((D, VB), lambda i: (0, HALF1_BASE // VB + i)),
        ],
        out_specs=pl.BlockSpec((VB, 128), lambda i: (i, 0)),
        out_shape=jax.ShapeDtypeStruct((NPAIR, 128), jnp.float32),
    )(tT, tT)


SLOTS = 8


def _conv2_body(x_ref, o_ref):
    bt = o_ref.shape[2]
    h = bt // 2
    for q in range(SLOTS):
        o_ref[q, :, 0:h] = x_ref[pl.ds(q * h, h), 0:D].T
        o_ref[q, :, h:bt] = x_ref[pl.ds(q * h, h), D : 2 * D].T


def _transpose_out(view, S, BT):
    return pl.pallas_call(
        _conv2_body,
        grid=(S // SLOTS,),
        in_specs=[
            pl.BlockSpec((SLOTS * BT // 2, 128), lambda s: (s, 0))
        ],
        out_specs=pl.BlockSpec((SLOTS, D, BT), lambda s: (s, 0, 0)),
        out_shape=jax.ShapeDtypeStruct((S, D, BT), jnp.float32),
    )(view)


@functools.partial(jax.jit, static_argnums=(0,))
def _gather_call(B, idx2d, table):
    info = plsc.get_sparse_core_info()
    nw = info.num_cores * info.num_subcores
    assert B % (nw * CHUNK) == 0
    n_chunks = B // (nw * CHUNK)
    mesh = plsc.VectorSubcoreMesh(core_axis_name="c", subcore_axis_name="s")

    @functools.partial(
        pl.kernel,
        mesh=mesh,
        out_type=jax.ShapeDtypeStruct((B, D), jnp.float32),
        scratch_types=[
            pltpu.VMEM((K, LANE), jnp.int32),
            pltpu.VMEM((CHUNK, D), jnp.float32),
            pltpu.SemaphoreType.DMA,
        ],
        compiler_params=pltpu.CompilerParams(use_tc_tiling_on_sc=False),
    )
    def gather_kernel(idx_hbm, table_hbm, out_hbm, idx_v, rows_v, sem):
        wid = lax.axis_index("s") * info.num_cores + lax.axis_index("c")
        idx_row0 = wid * (n_chunks * K)
        out_row0 = wid * (n_chunks * CHUNK)

        def chunk_body(i, carry):
            pltpu.sync_copy(idx_hbm.at[pl.ds(idx_row0 + i * K, K)], idx_v)
            for j in range(K):
                for k in range(LANE // 16):
                    v = idx_v[j, pl.ds(k * 16, 16)]
                    v2 = v + v
                    idx_v[j, pl.ds(k * 16, 16)] = jnp.where(
                        v < NPAIR, v2, v2 - (2 * HALF1_BASE - 1)
                    )
            cps = [
                pltpu.async_copy(
                    table_hbm.at[idx_v.at[j]],
                    rows_v.at[pl.ds(j * LANE, LANE)],
                    sem,
                )
                for j in range(K)
            ]
            for cp in cps:
                cp.wait()
            pltpu.sync_copy(
                rows_v, out_hbm.at[pl.ds(out_row0 + i * CHUNK, CHUNK)]
            )
            return carry

        lax.fori_loop(0, n_chunks, chunk_body, 0)

    return gather_kernel(idx2d, table)


def kernel(tokens, tok_embeddings):
    bt, s = tokens.shape
    B = bt * s
    table_rm = _repack_table(tok_embeddings.T.astype(jnp.float32))
    idx2d = (
        tokens.T.astype(jnp.int32)
        .reshape(s, 2, bt // 2)
        .swapaxes(1, 2)
        .reshape(B // LANE, LANE)
    )
    out_rm = _gather_call(B, idx2d, table_rm.reshape(2 * NPAIR, D))
    res = _transpose_out(out_rm.reshape(B // 2, 128), s, bt)
    return res.transpose(2, 0, 1)


# speedup vs baseline: 2.4447x; 1.0003x over previous
import functools

import jax
import jax.numpy as jnp
from jax import lax
from jax.experimental import pallas as pl
from jax.experimental.pallas import tpu as pltpu
from jax.experimental.pallas import tpu_sc as plsc

D = 64
LANE = 128
K = 8
CHUNK = K * LANE
VB = 8192
NPAIR = 62 * VB
HALF1_BASE = 61 * VB


def _conv1_body(x1_ref, x2_ref, o_ref):
    o_ref[:, 0:D] = x1_ref[...].T
    o_ref[:, D : 2 * D] = x2_ref[...].T


def _repack_table(tT):
    return pl.pallas_call(
        _conv1_body,
        grid=(NPAIR // VB,),
        in_specs=[
            pl.BlockSpec((D, VB), lambda i: (0, i)),
            pl.BlockSpec((D, VB), lambda i: (0, HALF1_BASE // VB + i)),
        ],
        out_specs=pl.BlockSpec((VB, 128), lambda i: (i, 0)),
        out_shape=jax.ShapeDtypeStruct((NPAIR, 128), jnp.float32),
    )(tT, tT)


SLOTS = 10


def _conv2_body(x_ref, o_ref):
    bt = o_ref.shape[2]
    h = bt // 2
    for q in range(SLOTS):
        o_ref[q, :, 0:h] = x_ref[pl.ds(q * h, h), 0:D].T
        o_ref[q, :, h:bt] = x_ref[pl.ds(q * h, h), D : 2 * D].T


def _transpose_out(view, S, BT):
    return pl.pallas_call(
        _conv2_body,
        grid=(S // SLOTS,),
        in_specs=[
            pl.BlockSpec((SLOTS * BT // 2, 128), lambda s: (s, 0))
        ],
        out_specs=pl.BlockSpec((SLOTS, D, BT), lambda s: (s, 0, 0)),
        out_shape=jax.ShapeDtypeStruct((S, D, BT), jnp.float32),
    )(view)


@functools.partial(jax.jit, static_argnums=(0,))
def _gather_call(B, idx2d, table):
    info = plsc.get_sparse_core_info()
    nw = info.num_cores * info.num_subcores
    assert B % (nw * CHUNK) == 0
    n_chunks = B // (nw * CHUNK)
    mesh = plsc.VectorSubcoreMesh(core_axis_name="c", subcore_axis_name="s")

    @functools.partial(
        pl.kernel,
        mesh=mesh,
        out_type=jax.ShapeDtypeStruct((B, D), jnp.float32),
        scratch_types=[
            pltpu.VMEM((K, LANE), jnp.int32),
            pltpu.VMEM((CHUNK, D), jnp.float32),
            pltpu.SemaphoreType.DMA,
        ],
        compiler_params=pltpu.CompilerParams(use_tc_tiling_on_sc=False),
    )
    def gather_kernel(idx_hbm, table_hbm, out_hbm, idx_v, rows_v, sem):
        wid = lax.axis_index("s") * info.num_cores + lax.axis_index("c")
        idx_row0 = wid * (n_chunks * K)
        out_row0 = wid * (n_chunks * CHUNK)

        def chunk_body(i, carry):
            pltpu.sync_copy(idx_hbm.at[pl.ds(idx_row0 + i * K, K)], idx_v)
            for j in range(K):
                for k in range(LANE // 16):
                    v = idx_v[j, pl.ds(k * 16, 16)]
                    v2 = v + v
                    idx_v[j, pl.ds(k * 16, 16)] = jnp.where(
                        v < NPAIR, v2, v2 - (2 * HALF1_BASE - 1)
                    )
            cps = [
                pltpu.async_copy(
                    table_hbm.at[idx_v.at[j]],
                    rows_v.at[pl.ds(j * LANE, LANE)],
                    sem,
                )
                for j in range(K)
            ]
            for cp in cps:
                cp.wait()
            pltpu.sync_copy(
                rows_v, out_hbm.at[pl.ds(out_row0 + i * CHUNK, CHUNK)]
            )
            return carry

        lax.fori_loop(0, n_chunks, chunk_body, 0)

    return gather_kernel(idx2d, table)


def kernel(tokens, tok_embeddings):
    bt, s = tokens.shape
    B = bt * s
    table_rm = _repack_table(tok_embeddings.T.astype(jnp.float32))
    idx2d = (
        tokens.T.astype(jnp.int32)
        .reshape(s, 2, bt // 2)
        .swapaxes(1, 2)
        .reshape(B // LANE, LANE)
    )
    out_rm = _gather_call(B, idx2d, table_rm.reshape(2 * NPAIR, D))
    res = _transpose_out(out_rm.reshape(B // 2, 128), s, bt)
    return res.transpose(2, 0, 1)
